# baseline (speedup 1.0000x reference)
import functools

import jax
import jax.numpy as jnp
from jax import lax
from jax.experimental import pallas as pl
from jax.experimental.pallas import tpu as pltpu
from jax.experimental.pallas import tpu_sc as plsc

B = 128; C = 512; H = 14; W = 14; HW = H * W
NFIND = 256; NROOT = 64; NANS = 1000
V = 5000; L = 20; DEMB = 300; DHID = 512; K = 2

TE = 8
DP = 384
VP = 5008
NW = 32
EX_W = B // NW
IDX_W = EX_W * L


def _softmax_rows(x):
    m = jnp.max(x, axis=-1, keepdims=True)
    e = jnp.exp(x - m)
    return e / jnp.sum(e, axis=-1, keepdims=True)


def _find_body(sf0, sf1, feat_ref, fw_ref, maps_ref, att_ref):
    i = pl.program_id(0)
    for e in range(TE):
        b = i * TE + e
        feat = feat_ref[e]
        w0 = fw_ref[pl.ds(sf0[b], 1), :]
        w1 = fw_ref[pl.ds(sf1[b], 1), :]
        w01 = jnp.concatenate([w0, w1], axis=0)
        att = jnp.maximum(
            jnp.dot(w01, feat, preferred_element_type=jnp.float32), 0.0)
        maps = att[0:1] * att[1:2]
        maps_ref[pl.ds(e, 1), :] = maps
        att_ref[pl.ds(e, 1), :] = lax.dot_general(
            maps, feat, (((1,), (1,)), ((), ())),
            preferred_element_type=jnp.float32)


def _find(features3, find_w, f0, f1):
    grid_spec = pltpu.PrefetchScalarGridSpec(
        num_scalar_prefetch=2,
        grid=(B // TE,),
        in_specs=[
            pl.BlockSpec((TE, C, HW), lambda i, a, b: (i, 0, 0)),
            pl.BlockSpec((NFIND, C), lambda i, a, b: (0, 0)),
        ],
        out_specs=[
            pl.BlockSpec((TE, HW), lambda i, a, b: (i, 0)),
            pl.BlockSpec((TE, C), lambda i, a, b: (i, 0)),
        ],
    )
    return pl.pallas_call(
        _find_body,
        grid_spec=grid_spec,
        out_shape=[jax.ShapeDtypeStruct((B, HW), jnp.float32),
                   jax.ShapeDtypeStruct((B, C), jnp.float32)],
        compiler_params=pltpu.CompilerParams(
            dimension_semantics=("arbitrary",)),
    )(f0, f1, features3, find_w)


def _expert_body(xflat_ref, xatt_ref, mw_ref, dw_ref, mb_ref, db_ref,
                 mm_ref, md_ref, out_ref):
    r = pl.program_id(0)
    ym = jnp.dot(xflat_ref[...], mw_ref[0],
                 preferred_element_type=jnp.float32) + mb_ref[0]
    yd = jnp.dot(xatt_ref[...], dw_ref[0],
                 preferred_element_type=jnp.float32) + db_ref[0]
    cur = out_ref[...]
    cur = jnp.where(mm_ref[0] > 0, ym, cur)
    cur = jnp.where(md_ref[0] > 0, yd, cur)
    out_ref[...] = cur

    @pl.when(r == NROOT - 1)
    def _softmax():
        out_ref[...] = _softmax_rows(out_ref[...])


def _experts(xflat, xatt, measure_w, describe_w, measure_b3, describe_b3,
             maskm, maskd):
    return pl.pallas_call(
        _expert_body,
        grid=(NROOT,),
        in_specs=[
            pl.BlockSpec((B, HW), lambda r: (0, 0)),
            pl.BlockSpec((B, C), lambda r: (0, 0)),
            pl.BlockSpec((1, HW, NANS), lambda r: (r, 0, 0)),
            pl.BlockSpec((1, C, NANS), lambda r: (r, 0, 0)),
            pl.BlockSpec((1, 1, NANS), lambda r: (r, 0, 0)),
            pl.BlockSpec((1, 1, NANS), lambda r: (r, 0, 0)),
            pl.BlockSpec((1, B, 1), lambda r: (r, 0, 0)),
            pl.BlockSpec((1, B, 1), lambda r: (r, 0, 0)),
        ],
        out_specs=pl.BlockSpec((B, NANS), lambda r: (0, 0)),
        out_shape=jax.ShapeDtypeStruct((B, NANS), jnp.float32),
        compiler_params=pltpu.CompilerParams(
            dimension_semantics=("arbitrary",)),
    )(xflat, xatt, measure_w, describe_w, measure_b3, describe_b3,
      maskm, maskd)


def _pool_sums_sc(qflat, emb_pad):
    mesh = plsc.VectorSubcoreMesh(core_axis_name="c", subcore_axis_name="s")

    @functools.partial(
        pl.kernel, mesh=mesh,
        out_type=jax.ShapeDtypeStruct((B, DP), jnp.float32),
        scratch_types=[
            pltpu.VMEM((IDX_W,), jnp.int32),
            pltpu.VMEM((IDX_W, DP), jnp.float32),
            pltpu.VMEM((EX_W, DP), jnp.float32),
            pltpu.SemaphoreType.DMA,
        ],
    )
    def k(q_hbm, emb_hbm, out_hbm, idx_v, rows_v, acc_v, sem):
        wid = lax.axis_index("s") * 2 + lax.axis_index("c")
        base = wid * IDX_W
        pltpu.sync_copy(q_hbm.at[pl.ds(base, IDX_W)], idx_v)
        pltpu.async_copy(emb_hbm.at[idx_v], rows_v, sem).wait()
        nj = DP // 16
        for e in range(EX_W):
            def body(t, carry):
                r = e * L + t
                return tuple(c + rows_v[r, pl.ds(j * 16, 16)]
                             for j, c in enumerate(carry))
            acc = lax.fori_loop(
                0, L, body,
                tuple(jnp.zeros((16,), jnp.float32) for _ in range(nj)))
            for j in range(nj):
                acc_v[e, pl.ds(j * 16, 16)] = acc[j]
        pltpu.sync_copy(acc_v, out_hbm.at[pl.ds(wid * EX_W, EX_W)])

    return k(qflat, emb_pad)


def _enc_body(sums_ref, len_ref, w1_ref, b1_ref, w2_ref, b2_ref, rp_ref,
              out_ref):
    pooled = sums_ref[...] / len_ref[...]
    h = jnp.tanh(jnp.dot(pooled, w1_ref[...],
                         preferred_element_type=jnp.float32) + b1_ref[...])
    logits = jnp.dot(h, w2_ref[...],
                     preferred_element_type=jnp.float32) + b2_ref[...]
    enc = _softmax_rows(logits)
    out_ref[...] = jnp.sqrt(rp_ref[...] * enc + 1e-30)


def _final(sums, lclip_f, enc_w1p, enc_b1, enc_w2, enc_b2, root_pred):
    return pl.pallas_call(
        _enc_body,
        out_shape=jax.ShapeDtypeStruct((B, NANS), jnp.float32),
    )(sums, lclip_f, enc_w1p, enc_b1, enc_w2, enc_b2, root_pred)


def kernel(features, question, length, yesno, root_inst, find_inst, find_w,
           measure_w, measure_b, describe_w, describe_b, emb, enc_w1, enc_b1,
           enc_w2, enc_b2):
    f32 = jnp.float32

    features3 = features.reshape(B, C, HW)
    measure_b3 = measure_b.reshape(NROOT, 1, NANS)
    describe_b3 = describe_b.reshape(NROOT, 1, NANS)
    emb_pad = jnp.zeros((VP, DP), f32).at[:V, :DEMB].set(emb)
    enc_w1p = jnp.zeros((DP, DHID), f32).at[:DEMB].set(enc_w1)
    b1r = enc_b1.reshape(1, DHID)
    b2r = enc_b2.reshape(1, NANS)

    rids = jnp.arange(NROOT, dtype=jnp.int32)[:, None]
    is_r = rids == root_inst.astype(jnp.int32)[None, :]
    maskm = (is_r & yesno[None, :]).astype(f32).reshape(NROOT, B, 1)
    maskd = (is_r & ~yesno[None, :]).astype(f32).reshape(NROOT, B, 1)
    f0 = find_inst[:, 0].astype(jnp.int32)
    f1 = find_inst[:, 1].astype(jnp.int32)

    lclip = jnp.clip(length, 1, L).astype(jnp.int32)
    qmask = jnp.arange(L, dtype=jnp.int32)[None, :] < lclip[:, None]
    qflat = jnp.where(qmask, question.astype(jnp.int32), V).reshape(-1)

    xflat, xatt = _find(features3, find_w, f0, f1)
    root_pred = _experts(xflat, xatt, measure_w, describe_w,
                         measure_b3, describe_b3, maskm, maskd)
    sums = _pool_sums_sc(qflat, emb_pad)
    out = _final(sums, lclip.astype(f32).reshape(B, 1), enc_w1p, b1r,
                 enc_w2, b2r, root_pred)
    return out
